# baseline (speedup 1.0000x reference)
import functools

import jax
import jax.numpy as jnp
from jax import lax
from jax.experimental import pallas as pl
from jax.experimental.pallas import tpu as pltpu
from jax.experimental.pallas import tpu_sc as plsc

B = 1024
L = 200
D = 128
NTOK = B * L
CHUNK = 80
NW = 32
CPW = NTOK // (NW * CHUNK)
NBUF = 8
GDEPTH = 4


def _build_kernel():
    mesh = plsc.VectorSubcoreMesh(core_axis_name="c", subcore_axis_name="s")
    info = plsc.get_sparse_core_info()
    nc = info.num_cores

    scratch = [pltpu.VMEM((CPW, CHUNK), jnp.int32)]
    scratch += [pltpu.VMEM((CHUNK, D), jnp.float32) for _ in range(NBUF)]
    scratch += [pltpu.SemaphoreType.DMA for _ in range(2 * NBUF)]

    @functools.partial(
        pl.kernel,
        out_type=jax.ShapeDtypeStruct((NTOK, D), jnp.float32),
        mesh=mesh,
        scratch_types=scratch,
    )
    def body(idx_hbm, emb_hbm, out_hbm, idx_v, *bufs_and_sems):
        bufs = bufs_and_sems[:NBUF]
        gsem = bufs_and_sems[NBUF:2 * NBUF]
        wsem = bufs_and_sems[2 * NBUF:]

        wid = lax.axis_index("s") * nc + lax.axis_index("c")
        base = wid * (CPW * CHUNK)
        pltpu.sync_copy(idx_hbm.at[wid], idx_v)
        for b in range(GDEPTH):
            pltpu.async_copy(emb_hbm.at[idx_v.at[b]], bufs[b], gsem[b])

        def group(i, carry):
            o = i * NBUF
            for b in range(NBUF):
                c = o + b
                nb = (b + GDEPTH) % NBUF
                pltpu.make_async_copy(
                    emb_hbm.at[idx_v.at[c]], bufs[b], gsem[b]).wait()
                pltpu.async_copy(
                    bufs[b], out_hbm.at[pl.ds(base + c * CHUNK, CHUNK)],
                    wsem[b])

                @pl.when(c + GDEPTH < CPW)
                def _(nb=nb, c=c):
                    @pl.when(c >= NBUF - GDEPTH)
                    def _():
                        pltpu.make_async_copy(
                            bufs[nb], out_hbm.at[pl.ds(base, CHUNK)],
                            wsem[nb]).wait()
                    pltpu.async_copy(
                        emb_hbm.at[idx_v.at[c + GDEPTH]], bufs[nb], gsem[nb])
            return carry

        lax.fori_loop(0, CPW // NBUF, group, 0)

        for b in range(NBUF):
            pltpu.make_async_copy(
                bufs[b], out_hbm.at[pl.ds(base, CHUNK)], wsem[b]).wait()

    return body


_kernel_fn = _build_kernel()


@jax.jit
def kernel(X, word_emb):
    idx = X.reshape(NW, CPW, CHUNK).astype(jnp.int32)
    out = _kernel_fn(idx, word_emb)
    return out.reshape(B, L, D)
